# Initial kernel scaffold; baseline (speedup 1.0000x reference)
import jax
import jax.numpy as jnp
from jax.experimental import pallas as pl


def kernel(input_, embed):
    raise NotImplementedError("write your pallas kernel here")


# speedup vs baseline: 1.6579x; 1.6579x over previous
import jax
import jax.numpy as jnp
from jax import lax
from jax.experimental import pallas as pl
from jax.experimental.pallas import tpu as pltpu

ROWS = 16384
DIM = 64
NCODES = 1024
TILE = 1024


def _vq_kernel(x_ref, e_ref, q_ref, ind_ref, dsum_ref):
    i = pl.program_id(0)
    x = x_ref[...]
    e = e_ref[...]
    xe = jnp.dot(x, e, preferred_element_type=jnp.float32)
    e2 = jnp.sum(e * e, axis=0, keepdims=True)
    x2 = jnp.sum(x * x, axis=1, keepdims=True)
    dist = x2 - 2.0 * xe + e2
    ind = jnp.argmin(dist, axis=1)
    onehot = (
        lax.broadcasted_iota(jnp.int32, (TILE, NCODES), 1) == ind[:, None]
    ).astype(jnp.float32)
    q = lax.dot_general(
        onehot, e, (((1,), (1,)), ((), ())),
        preferred_element_type=jnp.float32,
    )
    q_ref[...] = x + (q - x)
    ind_ref[...] = ind[:, None]

    @pl.when(i == 0)
    def _():
        dsum_ref[...] = jnp.zeros_like(dsum_ref)

    dsum_ref[...] += jnp.sum((q - x) ** 2, keepdims=True)


def kernel(input_, embed):
    grid = (ROWS // TILE,)
    q, ind, dsum = pl.pallas_call(
        _vq_kernel,
        grid=grid,
        in_specs=[
            pl.BlockSpec((TILE, DIM), lambda i: (i, 0)),
            pl.BlockSpec((DIM, NCODES), lambda i: (0, 0)),
        ],
        out_specs=[
            pl.BlockSpec((TILE, DIM), lambda i: (i, 0)),
            pl.BlockSpec((TILE, 1), lambda i: (i, 0)),
            pl.BlockSpec((1, 1), lambda i: (0, 0)),
        ],
        out_shape=[
            jax.ShapeDtypeStruct((ROWS, DIM), jnp.float32),
            jax.ShapeDtypeStruct((ROWS, 1), jnp.int32),
            jax.ShapeDtypeStruct((1, 1), jnp.float32),
        ],
    )(input_, embed)
    diff = dsum[0, 0] / (ROWS * DIM)
    return q, diff, ind.reshape(-1)


# speedup vs baseline: 1.8535x; 1.1180x over previous
import jax
import jax.numpy as jnp
from jax import lax
from jax.experimental import pallas as pl
from jax.experimental.pallas import tpu as pltpu

ROWS = 16384
DIM = 64
NCODES = 1024
TILE = 2048


def _vq_kernel(x_ref, e_ref, q_ref, ind_ref, dsum_ref):
    i = pl.program_id(0)
    x = x_ref[...]
    e = e_ref[...]
    xe = jnp.dot(x, e, preferred_element_type=jnp.float32)
    e2 = jnp.sum(e * e, axis=0, keepdims=True)
    x2 = jnp.sum(x * x, axis=1, keepdims=True)
    dist = x2 - 2.0 * xe + e2
    ind = jnp.argmin(dist, axis=1)
    onehot = (
        lax.broadcasted_iota(jnp.int32, (TILE, NCODES), 1) == ind[:, None]
    ).astype(jnp.float32)
    q = lax.dot_general(
        onehot, e, (((1,), (1,)), ((), ())),
        preferred_element_type=jnp.float32,
    )
    q_ref[...] = q
    ind_ref[...] = ind[:, None]

    @pl.when(i == 0)
    def _():
        dsum_ref[...] = jnp.zeros_like(dsum_ref)

    dsum_ref[...] += jnp.sum((q - x) ** 2, keepdims=True)


def kernel(input_, embed):
    grid = (ROWS // TILE,)
    q, ind, dsum = pl.pallas_call(
        _vq_kernel,
        grid=grid,
        in_specs=[
            pl.BlockSpec((TILE, DIM), lambda i: (i, 0)),
            pl.BlockSpec((DIM, NCODES), lambda i: (0, 0)),
        ],
        out_specs=[
            pl.BlockSpec((TILE, DIM), lambda i: (i, 0)),
            pl.BlockSpec((TILE, 1), lambda i: (i, 0)),
            pl.BlockSpec((1, 1), lambda i: (0, 0)),
        ],
        out_shape=[
            jax.ShapeDtypeStruct((ROWS, DIM), jnp.float32),
            jax.ShapeDtypeStruct((ROWS, 1), jnp.int32),
            jax.ShapeDtypeStruct((1, 1), jnp.float32),
        ],
    )(input_, embed)
    diff = dsum[0, 0] / (ROWS * DIM)
    return q, diff, ind.reshape(-1)


# speedup vs baseline: 1.8709x; 1.0094x over previous
import jax
import jax.numpy as jnp
from jax import lax
from jax.experimental import pallas as pl
from jax.experimental.pallas import tpu as pltpu

ROWS = 16384
DIM = 64
NCODES = 1024
TILE = 4096


def _vq_kernel(x_ref, e_ref, q_ref, ind_ref, dsum_ref):
    i = pl.program_id(0)
    x = x_ref[...]
    e = e_ref[...]
    xe = jnp.dot(x, e, preferred_element_type=jnp.float32)
    e2 = jnp.sum(e * e, axis=0, keepdims=True)
    x2 = jnp.sum(x * x, axis=1, keepdims=True)
    dist = x2 - 2.0 * xe + e2
    ind = jnp.argmin(dist, axis=1)
    onehot = (
        lax.broadcasted_iota(jnp.int32, (TILE, NCODES), 1) == ind[:, None]
    ).astype(jnp.float32)
    q = lax.dot_general(
        onehot, e, (((1,), (1,)), ((), ())),
        preferred_element_type=jnp.float32,
    )
    q_ref[...] = q
    ind_ref[...] = ind[:, None]

    @pl.when(i == 0)
    def _():
        dsum_ref[...] = jnp.zeros_like(dsum_ref)

    dsum_ref[...] += jnp.sum((q - x) ** 2, keepdims=True)


def kernel(input_, embed):
    grid = (ROWS // TILE,)
    q, ind, dsum = pl.pallas_call(
        _vq_kernel,
        grid=grid,
        in_specs=[
            pl.BlockSpec((TILE, DIM), lambda i: (i, 0)),
            pl.BlockSpec((DIM, NCODES), lambda i: (0, 0)),
        ],
        out_specs=[
            pl.BlockSpec((TILE, DIM), lambda i: (i, 0)),
            pl.BlockSpec((TILE, 1), lambda i: (i, 0)),
            pl.BlockSpec((1, 1), lambda i: (0, 0)),
        ],
        out_shape=[
            jax.ShapeDtypeStruct((ROWS, DIM), jnp.float32),
            jax.ShapeDtypeStruct((ROWS, 1), jnp.int32),
            jax.ShapeDtypeStruct((1, 1), jnp.float32),
        ],
    )(input_, embed)
    diff = dsum[0, 0] / (ROWS * DIM)
    return q, diff, ind.reshape(-1)
